# Initial kernel scaffold; baseline (speedup 1.0000x reference)
import jax
import jax.numpy as jnp
from jax.experimental import pallas as pl


def kernel(x, edge_index, W1, b1, W2, b2):
    raise NotImplementedError("write your pallas kernel here")


# speedup vs baseline: 1.0295x; 1.0295x over previous
import functools

import jax
import jax.numpy as jnp
from jax import lax
from jax.experimental import pallas as pl
from jax.experimental.pallas import tpu as pltpu
from jax.experimental.pallas import tpu_sc as plsc

N = 10000
D = 128
E = 320000
NC = 2
NS = 16
NW = NC * NS
CHUNK = 128
EPT = E // NW
CH = -(-EPT // CHUNK)
CH += CH % 2
EPT_PAD = CH * CHUNK
E_PAD = NW * EPT_PAD
N_PAD = 10240
RPT = N_PAD // NS
DEG_ROWS = 2 * N_PAD
DRPT = DEG_ROWS // NS
DEG_W = 16
RB = 400
GRID = N // RB

_mesh = plsc.VectorSubcoreMesh(core_axis_name="c", subcore_axis_name="s")


@functools.partial(
    pl.kernel,
    out_type=jax.ShapeDtypeStruct((NC * DEG_ROWS, DEG_W), jnp.float32),
    mesh=_mesh,
    scratch_types=[
        pltpu.VMEM((CHUNK,), jnp.int32),
        pltpu.VMEM((CHUNK,), jnp.int32),
        pltpu.VMEM((CHUNK, DEG_W), jnp.float32),
        pltpu.VMEM((CHUNK, DEG_W), jnp.float32),
        pltpu.VMEM_SHARED((DEG_ROWS, DEG_W), jnp.float32),
        pltpu.SemaphoreType.DMA,
    ],
)
def _deg_kernel(idx_hbm, iota_hbm, ones_hbm, zeros_hbm, out_hbm,
                idx_v, ridx_v, ones_v, stage_v, hist_sh, sem):
    c = lax.axis_index("c")
    s = lax.axis_index("s")
    wid = c * NS + s
    pltpu.sync_copy(zeros_hbm, stage_v)
    pltpu.sync_copy(ones_hbm, ones_v)

    @pl.loop(0, DRPT // CHUNK)
    def _zero(j):
        pltpu.sync_copy(iota_hbm.at[pl.ds(s * DRPT + j * CHUNK, CHUNK)],
                        ridx_v)
        pltpu.sync_copy(stage_v, hist_sh.at[ridx_v])

    plsc.subcore_barrier()

    @pl.loop(0, 2 * CH)
    def _hist(g):
        pltpu.sync_copy(
            idx_hbm.at[pl.ds((wid * 2 * CH + g) * CHUNK, CHUNK)], idx_v)
        pltpu.sync_copy(ones_v, hist_sh.at[idx_v], add=True)

    plsc.subcore_barrier()

    @pl.loop(0, DRPT // CHUNK)
    def _out(j):
        pltpu.sync_copy(iota_hbm.at[pl.ds(s * DRPT + j * CHUNK, CHUNK)],
                        ridx_v)
        pltpu.sync_copy(hist_sh.at[ridx_v], stage_v)
        pltpu.sync_copy(
            stage_v,
            out_hbm.at[pl.ds(c * DEG_ROWS + s * DRPT + j * CHUNK, CHUNK)])


GK = 8
NG = CH // GK


@functools.partial(
    pl.kernel,
    out_type=jax.ShapeDtypeStruct((NC * N_PAD, D), jnp.float32),
    mesh=_mesh,
    scratch_types=[
        pltpu.VMEM((CHUNK,), jnp.int32),
        pltpu.VMEM((CHUNK,), jnp.int32),
        pltpu.VMEM((CHUNK,), jnp.int32),
        pltpu.VMEM((CHUNK, D), jnp.float32),
        pltpu.VMEM((CHUNK, D), jnp.float32),
        pltpu.VMEM_SHARED((N_PAD, D), jnp.float32),
        pltpu.SemaphoreType.DMA,
        pltpu.SemaphoreType.DMA,
    ],
)
def _agg_kernel(h_hbm, src_hbm, dst_hbm, iota_hbm, zeros_hbm, out_hbm,
                src_v, dst_v, ridx_v, rows_a, rows_b, agg_sh, sem_a, sem_b):
    c = lax.axis_index("c")
    s = lax.axis_index("s")
    wid = c * NS + s
    pltpu.sync_copy(zeros_hbm, rows_a)

    @pl.loop(0, RPT // CHUNK)
    def _zero(j):
        pltpu.sync_copy(iota_hbm.at[pl.ds(s * RPT + j * CHUNK, CHUNK)],
                        ridx_v)
        pltpu.sync_copy(rows_a, agg_sh.at[ridx_v])

    plsc.subcore_barrier()

    @pl.loop(0, CH)
    def _edges(j):
        base = (wid * CH + j) * CHUNK
        pltpu.sync_copy(src_hbm.at[pl.ds(base, CHUNK)], src_v)
        pltpu.sync_copy(dst_hbm.at[pl.ds(base, CHUNK)], dst_v)
        pltpu.async_copy(h_hbm.at[src_v], rows_b, sem_a).wait()
        pltpu.sync_copy(rows_b, agg_sh.at[dst_v], add=True)

    plsc.subcore_barrier()

    @pl.loop(0, RPT // CHUNK)
    def _out(j):
        pltpu.sync_copy(iota_hbm.at[pl.ds(s * RPT + j * CHUNK, CHUNK)],
                        ridx_v)
        pltpu.sync_copy(agg_sh.at[ridx_v], rows_a)
        pltpu.sync_copy(
            rows_a,
            out_hbm.at[pl.ds(c * N_PAD + s * RPT + j * CHUNK, CHUNK)])


def _l1_body(x_ref, dop_ref, w_ref, o_ref):
    deg = dop_ref[0] + dop_ref[1]
    nsrc = lax.rsqrt(jnp.maximum(deg, 1.0))[:, :1]
    o_ref[...] = jnp.dot(x_ref[...] * nsrc, w_ref[...],
                         preferred_element_type=jnp.float32)


def _l2_body(ap_ref, dip_ref, dop_ref, b_ref, w_ref, o_ref):
    agg = ap_ref[0] + ap_ref[1]
    ndst = lax.rsqrt(jnp.maximum(dip_ref[0] + dip_ref[1], 1.0))[:, :1]
    a = jnp.maximum(agg * ndst + b_ref[...], 0.0)
    nsrc = lax.rsqrt(jnp.maximum(dop_ref[0] + dop_ref[1], 1.0))[:, :1]
    o_ref[...] = jnp.dot(a * nsrc, w_ref[...],
                         preferred_element_type=jnp.float32)


def _l3_body(ap_ref, dip_ref, b_ref, o_ref):
    agg = ap_ref[0] + ap_ref[1]
    ndst = lax.rsqrt(jnp.maximum(dip_ref[0] + dip_ref[1], 1.0))[:, :1]
    o_ref[...] = jnp.maximum(agg * ndst + b_ref[...], 0.0)


_ROW = pl.BlockSpec((RB, D), lambda i: (i, 0))
_PART = pl.BlockSpec((NC, RB, D), lambda i: (0, i, 0))
_DEGB = pl.BlockSpec((NC, RB, DEG_W), lambda i: (0, i, 0))
_WB = pl.BlockSpec((D, D), lambda i: (0, 0))
_BB = pl.BlockSpec((1, D), lambda i: (0, 0))
_OUT = jax.ShapeDtypeStruct((N, D), jnp.float32)


def kernel(x, edge_index, W1, b1, W2, b2):
    src = edge_index[0]
    dst = edge_index[1]
    pad = E_PAD - E
    srcp = jnp.concatenate([src, jnp.zeros((pad,), jnp.int32)])
    srcp = srcp.reshape(NW, CH, CHUNK)
    dstp = jnp.concatenate([dst, jnp.full((pad,), N_PAD - 1, jnp.int32)])
    dstp = dstp.reshape(NW, CH, CHUNK)
    deg_idx = jnp.concatenate([srcp, dstp + N_PAD], axis=1)
    ones16 = jnp.ones((CHUNK, DEG_W), jnp.float32)
    zeros16 = jnp.zeros((CHUNK, DEG_W), jnp.float32)
    zeros128 = jnp.zeros((CHUNK, D), jnp.float32)

    ones_e = jnp.ones((E,), jnp.float32)
    deg_out = jax.ops.segment_sum(ones_e, src, num_segments=N)
    deg_in = jax.ops.segment_sum(ones_e, dst, num_segments=N)

    def _mkdeg(dv):
        dv = jnp.pad(dv, (0, N_PAD - N))
        dv = jnp.broadcast_to(dv[None, :, None], (1, N_PAD, DEG_W))
        return jnp.concatenate([dv, jnp.zeros_like(dv)], axis=0)

    dop = _mkdeg(deg_out)
    dip = _mkdeg(deg_in)

    h1 = pl.pallas_call(
        _l1_body, grid=(GRID,),
        in_specs=[_ROW, _DEGB, _WB], out_specs=_ROW, out_shape=_OUT,
    )(x, dop, W1)

    def _jnp_agg(h):
        msg = jnp.take(h, src, axis=0)
        agg = jax.ops.segment_sum(msg, dst, num_segments=N)
        agg = jnp.pad(agg, ((0, N_PAD - N), (0, 0)))
        return jnp.stack([agg, jnp.zeros_like(agg)])

    aggp1 = _jnp_agg(h1)

    h2 = pl.pallas_call(
        _l2_body, grid=(GRID,),
        in_specs=[_PART, _DEGB, _DEGB, _BB, _WB], out_specs=_ROW,
        out_shape=_OUT,
    )(aggp1, dip, dop, b1.reshape(1, D), W2)

    aggp2 = _jnp_agg(h2)

    out = pl.pallas_call(
        _l3_body, grid=(GRID,),
        in_specs=[_PART, _DEGB, _BB], out_specs=_ROW, out_shape=_OUT,
    )(aggp2, dip, b2.reshape(1, D))
    return out
